# Initial kernel scaffold; baseline (speedup 1.0000x reference)
import jax
import jax.numpy as jnp
from jax.experimental import pallas as pl


def kernel(x, edge_index, edge_attr, W1, root1, b1, W2, root2, b2, W3, root3, b3, W4, root4, b4, W5, root5, b5, W6, root6, b6):
    raise NotImplementedError("write your pallas kernel here")


# speedup vs baseline: 4.4232x; 4.4232x over previous
import jax
import jax.numpy as jnp
from jax import lax
from jax.experimental import pallas as pl
from jax.experimental.pallas import tpu as pltpu
from jax.experimental.pallas import tpu_sc as plsc

_N = 10000
_E = 320000
_K = 27
_B = 256
_NCH = _E // _B
_NW = 32
_CH = [1, 8, 16, 32, 16, 8, 1]
_CP = 128

_EB = 32000
_CPB = _EB // _B
_NB = 1000


def _rowlen(cout):
    return max(128, 8 * cout)


def _basis_body(attr_ref, src_ref, gidx_ref, bw_ref):
    a = attr_ref[...]
    p = jnp.clip(a * 2.0, 0.0, 2.0 - 1e-6)
    f = jnp.floor(p)
    frac = p - f
    i0 = f.astype(jnp.int32)
    src = src_ref[...]
    cell = i0[0:1] + 2 * i0[1:2] + 4 * i0[2:3]
    gidx_ref[...] = src * 8 + cell
    ws = []
    for s in range(8):
        w = jnp.ones((1, _EB), jnp.float32)
        for d in range(3):
            bit = (s >> d) & 1
            w = w * (frac[d:d + 1] if bit else 1.0 - frac[d:d + 1])
        ws.append(w)
    bw_ref[...] = jnp.concatenate(ws, 0)


_basis_call = pl.pallas_call(
    _basis_body,
    grid=(_E // _EB,),
    in_specs=[
        pl.BlockSpec((3, _EB), lambda i: (0, i)),
        pl.BlockSpec((1, _EB), lambda i: (0, i)),
    ],
    out_specs=[
        pl.BlockSpec((1, _EB), lambda i: (0, i)),
        pl.BlockSpec((8, _EB), lambda i: (0, i)),
    ],
    out_shape=[
        jax.ShapeDtypeStruct((1, _E), jnp.int32),
        jax.ShapeDtypeStruct((8, _E), jnp.float32),
    ],
)


def _head_body(x_ref, wf_ref, root_ref, b_ref, y_ref, r_ref):
    h = x_ref[...]
    for c in range(8):
        y_ref[:, c, :] = jnp.dot(h, wf_ref[:, c, :],
                                 preferred_element_type=jnp.float32)
    r_ref[...] = jnp.dot(h, root_ref[...], preferred_element_type=jnp.float32) + b_ref[...]


def _mid_body(cin, p_ref, r_ref, wf_ref, root_ref, b_ref, y_ref, rn_ref):
    t = p_ref[0, :, :cin] + p_ref[1, :, :cin] + r_ref[...]
    h = jnp.where(t > 0, t, jnp.exp(t) - 1.0)
    for c in range(8):
        y_ref[:, c, :] = jnp.dot(h, wf_ref[:, c, :],
                                 preferred_element_type=jnp.float32)
    rn_ref[...] = jnp.dot(h, root_ref[...], preferred_element_type=jnp.float32) + b_ref[...]


def _tail_body(p_ref, r_ref, h_ref):
    t = p_ref[0, :, 0:1] + p_ref[1, :, 0:1] + r_ref[...]
    h_ref[...] = jnp.where(t > 0, t, jnp.exp(t) - 1.0)


def _full(shape):
    nd = len(shape)
    return pl.BlockSpec(shape, lambda i: (0,) * nd)


def _make_head(cout):
    rl = _rowlen(cout)
    return pl.pallas_call(
        _head_body,
        grid=(_N // _NB,),
        in_specs=[
            pl.BlockSpec((_NB, 1), lambda i: (i, 0)),
            _full((1, 8, rl)),
            _full((1, cout)),
            _full((1, cout)),
        ],
        out_specs=[
            pl.BlockSpec((_NB, 8, rl), lambda i: (i, 0, 0)),
            pl.BlockSpec((_NB, cout), lambda i: (i, 0)),
        ],
        out_shape=[
            jax.ShapeDtypeStruct((_N, 8, rl), jnp.float32),
            jax.ShapeDtypeStruct((_N, cout), jnp.float32),
        ],
    )


def _make_mid(cin, cout2):
    import functools
    rl = _rowlen(cout2)
    return pl.pallas_call(
        functools.partial(_mid_body, cin),
        grid=(_N // _NB,),
        in_specs=[
            pl.BlockSpec((2, _NB, _CP), lambda i: (0, i, 0)),
            pl.BlockSpec((_NB, cin), lambda i: (i, 0)),
            _full((cin, 8, rl)),
            _full((cin, cout2)),
            _full((1, cout2)),
        ],
        out_specs=[
            pl.BlockSpec((_NB, 8, rl), lambda i: (i, 0, 0)),
            pl.BlockSpec((_NB, cout2), lambda i: (i, 0)),
        ],
        out_shape=[
            jax.ShapeDtypeStruct((_N, 8, rl), jnp.float32),
            jax.ShapeDtypeStruct((_N, cout2), jnp.float32),
        ],
    )


_tail_call = pl.pallas_call(
    _tail_body,
    grid=(_N // _NB,),
    in_specs=[
        pl.BlockSpec((2, _NB, _CP), lambda i: (0, i, 0)),
        pl.BlockSpec((_NB, 1), lambda i: (i, 0)),
    ],
    out_specs=pl.BlockSpec((_NB, 1), lambda i: (i, 0)),
    out_shape=jax.ShapeDtypeStruct((_N, 1), jnp.float32),
)


_EB2 = 2560


def _msg_body(cout, rows_ref, bw_ref, out_ref):
    acc = jnp.zeros((_EB2, cout), jnp.float32)
    for s in range(8):
        acc = acc + bw_ref[:, s:s + 1] * rows_ref[:, s * cout:(s + 1) * cout]
    if cout < _CP:
        acc = jnp.concatenate(
            [acc, jnp.zeros((_EB2, _CP - cout), jnp.float32)], axis=1)
    out_ref[...] = acc.reshape(_EB2 // _B, _B, _CP)


def _make_msg(cout):
    import functools
    rl = _rowlen(cout)
    return pl.pallas_call(
        functools.partial(_msg_body, cout),
        grid=(_E // _EB2,),
        in_specs=[
            pl.BlockSpec((_EB2, rl), lambda i: (i, 0)),
            pl.BlockSpec((_EB2, 8), lambda i: (i, 0)),
        ],
        out_specs=pl.BlockSpec((_EB2 // _B, _B, _CP), lambda i: (i, 0, 0)),
        out_shape=jax.ShapeDtypeStruct((_NCH, _B, _CP), jnp.float32),
    )


def _make_sc(cout):
    mesh = plsc.VectorSubcoreMesh(core_axis_name="c", subcore_axis_name="s")
    rl = _rowlen(cout)
    nv = max(1, cout // 16)

    def body(msg_ref, dst_ref, out_ref,
             g0, g1, dstb, msg0, msg1, zbuf, agg, sem, sem2):
        cid = lax.axis_index("c")
        sid = lax.axis_index("s")
        w = cid * 16 + sid

        def zb(j, _):
            zbuf[j // 8, pl.ds((j % 8) * 16, 16)] = jnp.zeros((16,), jnp.float32)
            return _
        lax.fori_loop(0, 64 * _CP // 16, zb, 0)

        @pl.when(sid < 15)
        def _():
            for k in range(9):
                pltpu.sync_copy(zbuf, agg.at[pl.ds(sid * 632 + k * 64, 64), :])
            pltpu.sync_copy(zbuf.at[pl.ds(0, 56), :],
                            agg.at[pl.ds(sid * 632 + 576, 56), :])

        @pl.when(sid == 15)
        def _():
            for k in range(8):
                pltpu.sync_copy(zbuf, agg.at[pl.ds(9480 + k * 64, 64), :])
            pltpu.sync_copy(zbuf.at[pl.ds(0, 8), :], agg.at[pl.ds(9992, 8), :])
        plsc.subcore_barrier()

        nch = jnp.where(w < _NCH - (_NCH // _NW) * _NW, _NCH // _NW + 1,
                        _NCH // _NW)

        def chunk_body(i, _):
            ch = w + i * _NW
            pltpu.sync_copy(msg_ref.at[ch, pl.ds(0, 128), :], msg0)
            pltpu.sync_copy(msg_ref.at[ch, pl.ds(128, 128), :], msg1)
            pltpu.sync_copy(dst_ref.at[ch], dstb)
            for k in range(8):
                g0[pl.ds(k * 16, 16)] = dstb[0, pl.ds(k * 16, 16)]
                g1[pl.ds(k * 16, 16)] = dstb[1, pl.ds(k * 16, 16)]
            pltpu.async_copy(msg0, agg.at[g0], sem, add=True).wait()
            pltpu.async_copy(msg1, agg.at[g1], sem2, add=True).wait()
            return _

        lax.fori_loop(0, nch, chunk_body, 0)
        plsc.subcore_barrier()

        @pl.when(sid < 15)
        def _():
            for k in range(9):
                pltpu.sync_copy(agg.at[pl.ds(sid * 632 + k * 64, 64), :], zbuf)
                pltpu.sync_copy(zbuf, out_ref.at[cid, pl.ds(sid * 632 + k * 64, 64), :])
            pltpu.sync_copy(agg.at[pl.ds(sid * 632 + 576, 56), :],
                            zbuf.at[pl.ds(0, 56), :])
            pltpu.sync_copy(zbuf.at[pl.ds(0, 56), :],
                            out_ref.at[cid, pl.ds(sid * 632 + 576, 56), :])

        @pl.when(sid == 15)
        def _():
            for k in range(8):
                pltpu.sync_copy(agg.at[pl.ds(9480 + k * 64, 64), :], zbuf)
                pltpu.sync_copy(zbuf, out_ref.at[cid, pl.ds(9480 + k * 64, 64), :])
            pltpu.sync_copy(agg.at[pl.ds(9992, 8), :], zbuf.at[pl.ds(0, 8), :])
            pltpu.sync_copy(zbuf.at[pl.ds(0, 8), :],
                            out_ref.at[cid, pl.ds(9992, 8), :])

    return pl.kernel(
        body,
        out_type=jax.ShapeDtypeStruct((2, _N, _CP), jnp.float32),
        mesh=mesh,
        scratch_types=[
            pltpu.VMEM((128,), jnp.int32),
            pltpu.VMEM((128,), jnp.int32),
            pltpu.VMEM((8, 128), jnp.int32),
            pltpu.VMEM((128, _CP), jnp.float32),
            pltpu.VMEM((128, _CP), jnp.float32),
            pltpu.VMEM((64, _CP), jnp.float32),
            pltpu.VMEM_SHARED((_N, _CP), jnp.float32),
            pltpu.SemaphoreType.DMA,
            pltpu.SemaphoreType.DMA,
        ],
        compiler_params=pltpu.CompilerParams(needs_layout_passes=False),
    )


_sc_cache = {}


def _sc_layer(y8, gidx, bwt, dstc, cout):
    if cout not in _sc_cache:
        _sc_cache[cout] = _make_sc(cout)
    rl = _rowlen(cout)
    rows8 = y8.reshape(_N * 8, rl)[gidx]
    msgp = _make_msg(cout)(rows8, bwt)
    return _sc_cache[cout](msgp, dstc)


def _corner_w8(W, cout):
    cin = W.shape[1]
    rl = _rowlen(cout)
    cols = []
    for cell in range(8):
        blk = [W[sum((((cell >> d) & 1) + ((s >> d) & 1)) * 3 ** d
                    for d in range(3))] for s in range(8)]
        cellw = jnp.concatenate(blk, axis=1)
        if 8 * cout < rl:
            cellw = jnp.concatenate(
                [cellw, jnp.zeros((cin, rl - 8 * cout), jnp.float32)], axis=1)
        cols.append(cellw)
    return jnp.stack(cols, axis=1)


def kernel(x, edge_index, edge_attr, W1, root1, b1, W2, root2, b2, W3, root3,
           b3, W4, root4, b4, W5, root5, b5, W6, root6, b6):
    src = edge_index[0].reshape(1, _E)
    d2 = edge_index[1].reshape(_NCH, 2, 128)
    dstc = jnp.concatenate([d2, jnp.zeros((_NCH, 6, 128), jnp.int32)], axis=1)
    g1, bw8 = _basis_call(edge_attr.T, src)
    gidx = g1.reshape(_E)
    bwt = bw8.T

    params = [(W1, root1, b1), (W2, root2, b2), (W3, root3, b3),
              (W4, root4, b4), (W5, root5, b5), (W6, root6, b6)]

    y8, r = _make_head(_CH[1])(x, _corner_w8(W1, _CH[1]), root1,
                               b1.reshape(1, _CH[1]))
    for l in range(1, 7):
        cout = _CH[l]
        p = _sc_layer(y8, gidx, bwt, dstc, cout)
        if l < 6:
            w2, root2_, b2_ = params[l]
            cin2, cout2 = _CH[l], _CH[l + 1]
            y8, r = _make_mid(cin2, cout2)(p, r, _corner_w8(w2, cout2),
                                           root2_, b2_.reshape(1, cout2))
        else:
            h = _tail_call(p, r)
    return h.reshape(-1)
